# Initial kernel scaffold; baseline (speedup 1.0000x reference)
import jax
import jax.numpy as jnp
from jax.experimental import pallas as pl


def kernel(x, degrees, z_in, z_out):
    raise NotImplementedError("write your pallas kernel here")


# speedup vs baseline: 1.4494x; 1.4494x over previous
import functools

import jax
import jax.numpy as jnp
from jax import lax
from jax.experimental import pallas as pl
from jax.experimental.pallas import tpu as pltpu
from jax.experimental.pallas import tpu_sc as plsc

H = 128
CH = 128
NC = 2
NS = 16
NW = NC * NS
L = 16


def _sc_centrality(xf, d0, d1, zcat):
    nodes = xf.shape[0]
    assert nodes % CH == 0
    nchunks = nodes // CH
    niter = (nchunks + NW - 1) // NW
    zrows = zcat.shape[0]

    mesh = plsc.VectorSubcoreMesh(core_axis_name="c", subcore_axis_name="s")

    @functools.partial(
        pl.kernel,
        out_type=jax.ShapeDtypeStruct((nodes, H), jnp.float32),
        mesh=mesh,
        scratch_types=dict(
            d0b=pltpu.VMEM((CH,), jnp.int32),
            d1b=pltpu.VMEM((CH,), jnp.int32),
            iin=pltpu.VMEM((CH,), jnp.int32),
            iout=pltpu.VMEM((CH,), jnp.int32),
            xb=pltpu.VMEM((CH, H), jnp.float32),
            za=pltpu.VMEM((CH, H), jnp.float32),
            zb=pltpu.VMEM((CH, H), jnp.float32),
            sem_x=pltpu.SemaphoreType.DMA,
            sem_g=pltpu.SemaphoreType.DMA,
        ),
    )
    def k(x_hbm, d0_hbm, d1_hbm, z_hbm, out_hbm, *,
          d0b, d1b, iin, iout, xb, za, zb, sem_x, sem_g):
        wid = lax.axis_index("s") * NC + lax.axis_index("c")

        @pl.loop(0, niter)
        def chunk_loop(i):
            cid = wid + i * NW

            @pl.when(cid < nchunks)
            def _():
                base = cid * CH
                cx = pltpu.async_copy(x_hbm.at[pl.ds(base, CH), :], xb, sem_x)
                pltpu.sync_copy(d0_hbm.at[pl.ds(base, CH)], d0b)
                pltpu.sync_copy(d1_hbm.at[pl.ds(base, CH)], d1b)
                for g in range(CH // L):
                    s = pl.ds(g * L, L)
                    d0v = d0b[s]
                    d1v = d1b[s]
                    pad = d0v == -1
                    iin[s] = jnp.where(pad, 65, jnp.minimum(d1v, 64))
                    iout[s] = jnp.where(
                        pad, zrows - 1, jnp.minimum(d0v, 64) + 66)
                ca = pltpu.async_copy(z_hbm.at[iin], za, sem_g)
                cb = pltpu.async_copy(z_hbm.at[iout], zb, sem_g)
                cx.wait()
                ca.wait()
                cb.wait()

                @plsc.parallel_loop(0, CH, 1, unroll=2)
                def row(n):
                    for j in range(H // L):
                        cs = pl.ds(j * L, L)
                        plsc.addupdate(xb.at[n, cs], za[n, cs] + zb[n, cs])

                pltpu.sync_copy(xb, out_hbm.at[pl.ds(base, CH), :])

    return k(xf, d0, d1, zcat)


def kernel(x, degrees, z_in, z_out):
    B, N, Hdim = x.shape
    zero = jnp.zeros((1, Hdim), jnp.float32)
    zcat = jnp.concatenate(
        [z_in.astype(jnp.float32), zero, z_out.astype(jnp.float32), zero], 0)
    d0 = degrees[:, 0, :].reshape(-1).astype(jnp.int32)
    d1 = degrees[:, 1, :].reshape(-1).astype(jnp.int32)
    xf = x.reshape(-1, Hdim)
    out = _sc_centrality(xf, d0, d1, zcat)
    return out.reshape(B, N, Hdim)


# speedup vs baseline: 1.4675x; 1.0125x over previous
import functools

import jax
import jax.numpy as jnp
from jax import lax
from jax.experimental import pallas as pl
from jax.experimental.pallas import tpu as pltpu
from jax.experimental.pallas import tpu_sc as plsc

H = 128
CH = 128
NC = 2
NS = 16
NW = NC * NS
L = 16


def _sc_centrality(xf, d0, d1, zcat):
    nodes = xf.shape[0]
    assert nodes % CH == 0
    nchunks = nodes // CH
    niter = (nchunks + NW - 1) // NW
    zrows = zcat.shape[0]

    mesh = plsc.VectorSubcoreMesh(core_axis_name="c", subcore_axis_name="s")

    @functools.partial(
        pl.kernel,
        out_type=jax.ShapeDtypeStruct((nodes, H), jnp.float32),
        mesh=mesh,
        scratch_types=dict(
            d0b=pltpu.VMEM((CH,), jnp.int32),
            d1b=pltpu.VMEM((CH,), jnp.int32),
            iin=pltpu.VMEM((CH,), jnp.int32),
            iout=pltpu.VMEM((CH,), jnp.int32),
            xb=pltpu.VMEM((CH, H), jnp.float32),
            za=pltpu.VMEM((CH, H), jnp.float32),
            zb=pltpu.VMEM((CH, H), jnp.float32),
            sem_x=pltpu.SemaphoreType.DMA,
            sem_g=pltpu.SemaphoreType.DMA,
        ),
    )
    def k(x_hbm, d0_hbm, d1_hbm, z_hbm, out_hbm, *,
          d0b, d1b, iin, iout, xb, za, zb, sem_x, sem_g):
        wid = lax.axis_index("s") * NC + lax.axis_index("c")

        @pl.loop(0, niter)
        def chunk_loop(i):
            cid = wid + i * NW

            @pl.when(cid < nchunks)
            def _():
                base = cid * CH
                cx = pltpu.async_copy(x_hbm.at[pl.ds(base, CH), :], xb, sem_x)
                pltpu.sync_copy(d0_hbm.at[pl.ds(base, CH)], d0b)
                pltpu.sync_copy(d1_hbm.at[pl.ds(base, CH)], d1b)
                for g in range(CH // L):
                    s = pl.ds(g * L, L)
                    d0v = d0b[s]
                    d1v = d1b[s]
                    pad = d0v == -1
                    iin[s] = jnp.where(pad, 65, jnp.minimum(d1v, 64))
                    iout[s] = jnp.where(
                        pad, zrows - 1, jnp.minimum(d0v, 64) + 66)
                cx.wait()
                ca = pltpu.async_copy(z_hbm.at[iin], xb, sem_g, add=True)
                cb = pltpu.async_copy(z_hbm.at[iout], xb, sem_g, add=True)
                ca.wait()
                cb.wait()

                pltpu.sync_copy(xb, out_hbm.at[pl.ds(base, CH), :])

    return k(xf, d0, d1, zcat)


def kernel(x, degrees, z_in, z_out):
    B, N, Hdim = x.shape
    zero = jnp.zeros((1, Hdim), jnp.float32)
    zcat = jnp.concatenate(
        [z_in.astype(jnp.float32), zero, z_out.astype(jnp.float32), zero], 0)
    d0 = degrees[:, 0, :].reshape(-1).astype(jnp.int32)
    d1 = degrees[:, 1, :].reshape(-1).astype(jnp.int32)
    xf = x.reshape(-1, Hdim)
    out = _sc_centrality(xf, d0, d1, zcat)
    return out.reshape(B, N, Hdim)
